# Initial kernel scaffold; baseline (speedup 1.0000x reference)
import jax
import jax.numpy as jnp
from jax.experimental import pallas as pl


def kernel(positions, rotations, opacities, view_matrices):
    raise NotImplementedError("write your pallas kernel here")


# speedup vs baseline: 19.6433x; 19.6433x over previous
import jax
import jax.numpy as jnp
from jax.experimental import pallas as pl
from jax.experimental.pallas import tpu as pltpu

_N = 8192
_R = 256
_K = 10


def _normals_from_quat_rows(w, x, y, z):
    n2 = w * w + x * x + y * y + z * z
    inv = 1.0 / jnp.maximum(jnp.sqrt(n2), 1e-12)
    w, x, y, z = w * inv, x * inv, y * inv, z * inv
    nx = 2.0 * (x * z + w * y)
    ny = 2.0 * (y * z - w * x)
    nz = 1.0 - 2.0 * (x * x + y * y)
    return nx, ny, nz


def _body(p_ref, q_ref, o_ref, pt_ref, qt_ref, out_ref):
    i = pl.program_id(0)

    p = p_ref[...]
    pt = pt_ref[...]
    sq_r = jnp.sum(p * p, axis=1, keepdims=True)
    sq_c = jnp.sum(pt * pt, axis=0, keepdims=True)
    dots = jax.lax.dot_general(
        p.astype(jnp.bfloat16), pt.astype(jnp.bfloat16),
        (((1,), (0,)), ((), ())),
        preferred_element_type=jnp.float32)
    d2 = sq_r + sq_c - 2.0 * dots

    col = jax.lax.broadcasted_iota(jnp.int32, (_R, _N), 1)
    inf = jnp.float32(jnp.inf)
    d2 = jnp.maximum(d2, 0.0)

    m = jnp.min(d2, axis=1, keepdims=True)
    idx0 = jnp.min(jnp.where(d2 == m, col, _N), axis=1, keepdims=True)
    d2 = jnp.where(col == idx0, inf, d2)
    for _ in range(_K):
        m = jnp.min(d2, axis=1, keepdims=True)
        idx = jnp.min(jnp.where(d2 == m, col, _N), axis=1, keepdims=True)
        d2 = jnp.where(col == idx, inf, d2)

    sel = jnp.isinf(d2) & (col != idx0)

    nw, nx_, ny_, nz_ = (qt_ref[0:1, :], qt_ref[1:2, :],
                         qt_ref[2:3, :], qt_ref[3:4, :])
    anx, any_, anz = _normals_from_quat_rows(nw, nx_, ny_, nz_)

    zero = jnp.float32(0.0)
    mx = jnp.sum(jnp.where(sel, anx, zero), axis=1)
    my = jnp.sum(jnp.where(sel, any_, zero), axis=1)
    mz = jnp.sum(jnp.where(sel, anz, zero), axis=1)

    q = q_ref[...]
    onx, ony, onz = _normals_from_quat_rows(
        q[:, 0], q[:, 1], q[:, 2], q[:, 3])

    cos_sum = onx * mx + ony * my + onz * mz
    opac = o_ref[0, 0, :]
    partial = jnp.sum(opac * (1.0 - cos_sum * (1.0 / _K))).reshape(1, 1)

    @pl.when(i == 0)
    def _():
        out_ref[...] = jnp.zeros((1, 1), jnp.float32)
    out_ref[...] += partial


def kernel(positions, rotations, opacities, view_matrices):
    del view_matrices
    n = positions.shape[0]
    ppad = jnp.concatenate(
        [positions, jnp.zeros((n, 5), jnp.float32)], axis=1)
    qpad = jnp.concatenate(
        [rotations, jnp.zeros((n, 4), jnp.float32)], axis=1)
    pt = ppad.T
    qt = qpad.T
    opac = opacities.reshape(n // _R, 1, _R)

    grid = n // _R
    out = pl.pallas_call(
        _body,
        grid=(grid,),
        in_specs=[
            pl.BlockSpec((_R, 8), lambda i: (i, 0)),
            pl.BlockSpec((_R, 8), lambda i: (i, 0)),
            pl.BlockSpec((1, 1, _R), lambda i: (i, 0, 0)),
            pl.BlockSpec((8, _N), lambda i: (0, 0)),
            pl.BlockSpec((8, _N), lambda i: (0, 0)),
        ],
        out_specs=pl.BlockSpec((1, 1), lambda i: (0, 0)),
        out_shape=jax.ShapeDtypeStruct((1, 1), jnp.float32),
    )(ppad, qpad, opac, pt, qt)
    return out[0, 0] / n


# speedup vs baseline: 27.1933x; 1.3844x over previous
import jax
import jax.numpy as jnp
from jax.experimental import pallas as pl
from jax.experimental.pallas import tpu as pltpu

_N = 8192
_R = 256
_K = 10


def _normals_from_quat_rows(w, x, y, z):
    n2 = w * w + x * x + y * y + z * z
    inv = 1.0 / jnp.maximum(jnp.sqrt(n2), 1e-12)
    w, x, y, z = w * inv, x * inv, y * inv, z * inv
    nx = 2.0 * (x * z + w * y)
    ny = 2.0 * (y * z - w * x)
    nz = 1.0 - 2.0 * (x * x + y * y)
    return nx, ny, nz


def _body(p_ref, q_ref, o_ref, pt_ref, qt_ref, out_ref):
    i = pl.program_id(0)

    p = p_ref[...]
    pt = pt_ref[...]
    sq_r = jnp.sum(p * p, axis=1, keepdims=True)
    sq_c = jnp.sum(pt * pt, axis=0, keepdims=True)
    dots = jax.lax.dot_general(
        p.astype(jnp.bfloat16), pt.astype(jnp.bfloat16),
        (((1,), (0,)), ((), ())),
        preferred_element_type=jnp.float32)
    d2 = sq_r + sq_c - 2.0 * dots

    col = jax.lax.broadcasted_iota(jnp.int32, (_R, _N), 1)
    bits = jax.lax.bitcast_convert_type(jnp.maximum(d2, 0.0), jnp.int32)
    keys = (bits & jnp.int32(~0x1FFF)) | col
    big = jnp.int32(0x7FFFFFFF)

    m0 = jnp.min(keys, axis=1, keepdims=True)
    keys = jnp.where(keys == m0, big, keys)
    for _ in range(_K):
        m = jnp.min(keys, axis=1, keepdims=True)
        keys = jnp.where(keys == m, big, keys)

    idx0 = m0 & jnp.int32(0x1FFF)
    sel = (keys == big) & (col != idx0)

    nw, nx_, ny_, nz_ = (qt_ref[0:1, :], qt_ref[1:2, :],
                         qt_ref[2:3, :], qt_ref[3:4, :])
    anx, any_, anz = _normals_from_quat_rows(nw, nx_, ny_, nz_)

    zero = jnp.float32(0.0)
    mx = jnp.sum(jnp.where(sel, anx, zero), axis=1)
    my = jnp.sum(jnp.where(sel, any_, zero), axis=1)
    mz = jnp.sum(jnp.where(sel, anz, zero), axis=1)

    q = q_ref[...]
    onx, ony, onz = _normals_from_quat_rows(
        q[:, 0], q[:, 1], q[:, 2], q[:, 3])

    cos_sum = onx * mx + ony * my + onz * mz
    opac = o_ref[0, 0, :]
    partial = jnp.sum(opac * (1.0 - cos_sum * (1.0 / _K))).reshape(1, 1)

    @pl.when(i == 0)
    def _():
        out_ref[...] = jnp.zeros((1, 1), jnp.float32)
    out_ref[...] += partial


def kernel(positions, rotations, opacities, view_matrices):
    del view_matrices
    n = positions.shape[0]
    ppad = jnp.concatenate(
        [positions, jnp.zeros((n, 5), jnp.float32)], axis=1)
    qpad = jnp.concatenate(
        [rotations, jnp.zeros((n, 4), jnp.float32)], axis=1)
    pt = ppad.T
    qt = qpad.T
    opac = opacities.reshape(n // _R, 1, _R)

    grid = n // _R
    out = pl.pallas_call(
        _body,
        grid=(grid,),
        in_specs=[
            pl.BlockSpec((_R, 8), lambda i: (i, 0)),
            pl.BlockSpec((_R, 8), lambda i: (i, 0)),
            pl.BlockSpec((1, 1, _R), lambda i: (i, 0, 0)),
            pl.BlockSpec((8, _N), lambda i: (0, 0)),
            pl.BlockSpec((8, _N), lambda i: (0, 0)),
        ],
        out_specs=pl.BlockSpec((1, 1), lambda i: (0, 0)),
        out_shape=jax.ShapeDtypeStruct((1, 1), jnp.float32),
    )(ppad, qpad, opac, pt, qt)
    return out[0, 0] / n


# speedup vs baseline: 34.1300x; 1.2551x over previous
import functools

import jax
import jax.numpy as jnp
from jax import lax
from jax.experimental import pallas as pl
from jax.experimental.pallas import tpu as pltpu
from jax.experimental.pallas import tpu_sc as plsc

_N = 8192
_R = 256
_K = 10
_NW = 32
_RW = _N // _NW
_G = _RW // 16


def _normals_from_quat_rows(w, x, y, z):
    n2 = w * w + x * x + y * y + z * z
    inv = 1.0 / jnp.maximum(jnp.sqrt(n2), 1e-12)
    w, x, y, z = w * inv, x * inv, y * inv, z * inv
    nx = 2.0 * (x * z + w * y)
    ny = 2.0 * (y * z - w * x)
    nz = 1.0 - 2.0 * (x * x + y * y)
    return nx, ny, nz


def _tc_body(p_ref, pt_ref, qt_ref, idx_ref, nrm_ref):
    i = pl.program_id(0)

    p = p_ref[...]
    pt = pt_ref[...]
    sq_r = jnp.sum(p * p, axis=1, keepdims=True)
    sq_c = jnp.sum(pt * pt, axis=0, keepdims=True)
    dots = lax.dot_general(
        p.astype(jnp.bfloat16), pt.astype(jnp.bfloat16),
        (((1,), (0,)), ((), ())),
        preferred_element_type=jnp.float32)
    d2 = sq_r + sq_c - 2.0 * dots

    col = lax.broadcasted_iota(jnp.int32, (_R, _N), 1)
    bits = lax.bitcast_convert_type(jnp.maximum(d2, 0.0), jnp.int32)
    keys = (bits & jnp.int32(~0x1FFF)) | col
    big = jnp.int32(0x7FFFFFFF)

    lane16 = lax.broadcasted_iota(jnp.int32, (_R, 16), 1)
    acc_idx = jnp.zeros((_R, 16), jnp.int32)

    m0 = jnp.min(keys, axis=1, keepdims=True)
    keys = jnp.where(keys == m0, big, keys)
    for j in range(_K):
        m = jnp.min(keys, axis=1, keepdims=True)
        keys = jnp.where(keys == m, big, keys)
        acc_idx = jnp.where(lane16 == j, m & jnp.int32(0x1FFF), acc_idx)

    idx_ref[...] = acc_idx

    @pl.when(i == 0)
    def _():
        nw, nx_, ny_, nz_ = (qt_ref[0:1, :], qt_ref[1:2, :],
                             qt_ref[2:3, :], qt_ref[3:4, :])
        anx, any_, anz = _normals_from_quat_rows(nw, nx_, ny_, nz_)
        nrm_ref[0:1, :] = anx
        nrm_ref[1:2, :] = any_
        nrm_ref[2:3, :] = anz
        nrm_ref[3:8, :] = jnp.zeros((5, _N), jnp.float32)


def _tc_stage(ppad, pt, qt):
    grid = _N // _R
    return pl.pallas_call(
        _tc_body,
        grid=(grid,),
        in_specs=[
            pl.BlockSpec((_R, 8), lambda i: (i, 0)),
            pl.BlockSpec((8, _N), lambda i: (0, 0)),
            pl.BlockSpec((8, _N), lambda i: (0, 0)),
        ],
        out_specs=[
            pl.BlockSpec((_R, 16), lambda i: (i, 0)),
            pl.BlockSpec((8, _N), lambda i: (0, 0)),
        ],
        out_shape=[
            jax.ShapeDtypeStruct((_N, 16), jnp.int32),
            jax.ShapeDtypeStruct((8, _N), jnp.float32),
        ],
    )(ppad, pt, qt)


def _sc_stage(idx_all, nrm, opac):
    mesh = plsc.VectorSubcoreMesh(core_axis_name="c", subcore_axis_name="s")

    @functools.partial(
        pl.kernel, mesh=mesh,
        compiler_params=pltpu.CompilerParams(needs_layout_passes=False),
        out_type=jax.ShapeDtypeStruct((_NW, 16), jnp.float32),
        scratch_types=[
            pltpu.VMEM((_RW, 16), jnp.int32),
            pltpu.VMEM((_N,), jnp.float32),
            pltpu.VMEM((_N,), jnp.float32),
            pltpu.VMEM((_N,), jnp.float32),
            pltpu.VMEM((_RW,), jnp.float32),
            pltpu.VMEM((16,), jnp.float32),
        ],
    )
    def sc_loss(idx_hbm, nrm_hbm, op_hbm, out_hbm,
                idx_v, nx_v, ny_v, nz_v, op_v, acc_v):
        c = lax.axis_index("c")
        s = lax.axis_index("s")
        wid = s * 2 + c
        base = pl.multiple_of(wid * _RW, _RW)
        pltpu.sync_copy(idx_hbm.at[pl.ds(base, _RW)], idx_v)
        pltpu.sync_copy(nrm_hbm.at[0], nx_v)
        pltpu.sync_copy(nrm_hbm.at[1], ny_v)
        pltpu.sync_copy(nrm_hbm.at[2], nz_v)
        pltpu.sync_copy(op_hbm.at[pl.ds(base, _RW)], op_v)

        lanes = lax.iota(jnp.int32, 16)
        total = jnp.zeros((16,), jnp.float32)
        for g in range(_G):
            rows = g * 16 + lanes
            own = base + rows
            onx = plsc.load_gather(nx_v, [own])
            ony = plsc.load_gather(ny_v, [own])
            onz = plsc.load_gather(nz_v, [own])
            mx = jnp.zeros((16,), jnp.float32)
            my = jnp.zeros((16,), jnp.float32)
            mz = jnp.zeros((16,), jnp.float32)
            for j in range(_K):
                nid = plsc.load_gather(idx_v, [rows, jnp.full((16,), j, jnp.int32)])
                mx = mx + plsc.load_gather(nx_v, [nid])
                my = my + plsc.load_gather(ny_v, [nid])
                mz = mz + plsc.load_gather(nz_v, [nid])
            cos_sum = onx * mx + ony * my + onz * mz
            opv = op_v[pl.ds(g * 16, 16)]
            total = total + opv * (1.0 - cos_sum * (1.0 / _K))
        acc_v[...] = total
        pltpu.sync_copy(acc_v, out_hbm.at[wid])

    return sc_loss(idx_all, nrm, opac)


def kernel(positions, rotations, opacities, view_matrices):
    del view_matrices
    n = positions.shape[0]
    ppad = jnp.concatenate(
        [positions, jnp.zeros((n, 5), jnp.float32)], axis=1)
    qpad = jnp.concatenate(
        [rotations, jnp.zeros((n, 4), jnp.float32)], axis=1)
    pt = ppad.T
    qt = qpad.T

    idx_all, nrm = _tc_stage(ppad, pt, qt)
    partials = _sc_stage(idx_all, nrm, opacities.reshape(n))
    return jnp.sum(partials) / n


# speedup vs baseline: 47.5439x; 1.3930x over previous
import functools

import jax
import jax.numpy as jnp
from jax import lax
from jax.experimental import pallas as pl
from jax.experimental.pallas import tpu as pltpu
from jax.experimental.pallas import tpu_sc as plsc

_N = 8192
_R = 256
_K = 10
_NW = 32
_RW = _N // _NW
_G = _RW // 16


def _normals_from_quat_rows(w, x, y, z):
    n2 = w * w + x * x + y * y + z * z
    inv = 1.0 / jnp.maximum(jnp.sqrt(n2), 1e-12)
    w, x, y, z = w * inv, x * inv, y * inv, z * inv
    nx = 2.0 * (x * z + w * y)
    ny = 2.0 * (y * z - w * x)
    nz = 1.0 - 2.0 * (x * x + y * y)
    return nx, ny, nz


def _tc_body(p_ref, pb_ref, ptb_ref, pt_ref, qt_ref, idx_ref, nrm_ref):
    i = pl.program_id(0)

    p = p_ref[...]
    pt = pt_ref[...]
    sq_r = jnp.sum(p * p, axis=1, keepdims=True)
    sq_c = jnp.sum(pt * pt, axis=0, keepdims=True)
    dots = lax.dot_general(
        pb_ref[...], ptb_ref[...], (((1,), (0,)), ((), ())),
        preferred_element_type=jnp.float32)
    d2 = sq_r + sq_c - 2.0 * dots

    col = lax.broadcasted_iota(jnp.int32, (_R, _N), 1)
    bits = lax.bitcast_convert_type(jnp.maximum(d2, 0.0), jnp.int32)
    keys = lax.bitcast_convert_type(
        (bits & jnp.int32(~0x1FFF)) + (col + jnp.int32(0x00800000)),
        jnp.float32)
    inf = jnp.float32(jnp.inf)

    lane16 = lax.broadcasted_iota(jnp.int32, (_R, 16), 1)
    acc_idx = jnp.zeros((_R, 16), jnp.int32)

    m0 = jnp.min(keys, axis=1, keepdims=True)
    keys = jnp.where(keys == m0, inf, keys)
    for j in range(_K):
        m = jnp.min(keys, axis=1, keepdims=True)
        keys = jnp.where(keys == m, inf, keys)
        mi = lax.bitcast_convert_type(m, jnp.int32) & jnp.int32(0x1FFF)
        acc_idx = jnp.where(lane16 == j, mi, acc_idx)

    idx_ref[...] = acc_idx

    @pl.when(i == 0)
    def _():
        nw, nx_, ny_, nz_ = (qt_ref[0:1, :], qt_ref[1:2, :],
                             qt_ref[2:3, :], qt_ref[3:4, :])
        anx, any_, anz = _normals_from_quat_rows(nw, nx_, ny_, nz_)
        nrm_ref[0:1, :] = anx
        nrm_ref[1:2, :] = any_
        nrm_ref[2:3, :] = anz
        nrm_ref[3:8, :] = jnp.zeros((5, _N), jnp.float32)


def _tc_stage(ppad, pb, ptb, pt, qt):
    grid = _N // _R
    return pl.pallas_call(
        _tc_body,
        grid=(grid,),
        in_specs=[
            pl.BlockSpec((_R, 8), lambda i: (i, 0)),
            pl.BlockSpec((_R, 8), lambda i: (i, 0)),
            pl.BlockSpec((8, _N), lambda i: (0, 0)),
            pl.BlockSpec((8, _N), lambda i: (0, 0)),
            pl.BlockSpec((8, _N), lambda i: (0, 0)),
        ],
        out_specs=[
            pl.BlockSpec((_R, 16), lambda i: (i, 0)),
            pl.BlockSpec((8, _N), lambda i: (0, 0)),
        ],
        out_shape=[
            jax.ShapeDtypeStruct((_N, 16), jnp.int32),
            jax.ShapeDtypeStruct((8, _N), jnp.float32),
        ],
    )(ppad, pb, ptb, pt, qt)


def _sc_stage(idx_all, nrm, opac):
    mesh = plsc.VectorSubcoreMesh(core_axis_name="c", subcore_axis_name="s")

    @functools.partial(
        pl.kernel, mesh=mesh,
        compiler_params=pltpu.CompilerParams(needs_layout_passes=False),
        out_type=jax.ShapeDtypeStruct((_NW, 16), jnp.float32),
        scratch_types=[
            pltpu.VMEM((_RW, 16), jnp.int32),
            pltpu.VMEM((_N,), jnp.float32),
            pltpu.VMEM((_N,), jnp.float32),
            pltpu.VMEM((_N,), jnp.float32),
            pltpu.VMEM((_RW,), jnp.float32),
            pltpu.VMEM((16,), jnp.float32),
        ],
    )
    def sc_loss(idx_hbm, nrm_hbm, op_hbm, out_hbm,
                idx_v, nx_v, ny_v, nz_v, op_v, acc_v):
        c = lax.axis_index("c")
        s = lax.axis_index("s")
        wid = s * 2 + c
        base = pl.multiple_of(wid * _RW, _RW)
        pltpu.sync_copy(idx_hbm.at[pl.ds(base, _RW)], idx_v)
        pltpu.sync_copy(nrm_hbm.at[0], nx_v)
        pltpu.sync_copy(nrm_hbm.at[1], ny_v)
        pltpu.sync_copy(nrm_hbm.at[2], nz_v)
        pltpu.sync_copy(op_hbm.at[pl.ds(base, _RW)], op_v)

        lanes = lax.iota(jnp.int32, 16)
        total = jnp.zeros((16,), jnp.float32)
        for g in range(_G):
            rows = g * 16 + lanes
            own = base + rows
            onx = plsc.load_gather(nx_v, [own])
            ony = plsc.load_gather(ny_v, [own])
            onz = plsc.load_gather(nz_v, [own])
            mx = jnp.zeros((16,), jnp.float32)
            my = jnp.zeros((16,), jnp.float32)
            mz = jnp.zeros((16,), jnp.float32)
            for j in range(_K):
                nid = plsc.load_gather(idx_v, [rows, jnp.full((16,), j, jnp.int32)])
                mx = mx + plsc.load_gather(nx_v, [nid])
                my = my + plsc.load_gather(ny_v, [nid])
                mz = mz + plsc.load_gather(nz_v, [nid])
            cos_sum = onx * mx + ony * my + onz * mz
            opv = op_v[pl.ds(g * 16, 16)]
            total = total + opv * (1.0 - cos_sum * (1.0 / _K))
        acc_v[...] = total
        pltpu.sync_copy(acc_v, out_hbm.at[wid])

    return sc_loss(idx_all, nrm, opac)


def kernel(positions, rotations, opacities, view_matrices):
    del view_matrices
    n = positions.shape[0]
    ppad = jnp.concatenate(
        [positions, jnp.zeros((n, 5), jnp.float32)], axis=1)
    qpad = jnp.concatenate(
        [rotations, jnp.zeros((n, 4), jnp.float32)], axis=1)
    pt = ppad.T
    qt = qpad.T
    pb = ppad.astype(jnp.bfloat16)
    ptb = pt.astype(jnp.bfloat16)

    idx_all, nrm = _tc_stage(ppad, pb, ptb, pt, qt)
    partials = _sc_stage(idx_all, nrm, opacities.reshape(n))
    return jnp.sum(partials) / n


# speedup vs baseline: 48.8022x; 1.0265x over previous
import functools

import jax
import jax.numpy as jnp
from jax import lax
from jax.experimental import pallas as pl
from jax.experimental.pallas import tpu as pltpu
from jax.experimental.pallas import tpu_sc as plsc

_N = 8192
_R = 512
_K = 10
_NW = 32
_RW = _N // _NW
_G = _RW // 16


def _normals_from_quat_rows(w, x, y, z):
    n2 = w * w + x * x + y * y + z * z
    inv = 1.0 / jnp.maximum(jnp.sqrt(n2), 1e-12)
    w, x, y, z = w * inv, x * inv, y * inv, z * inv
    nx = 2.0 * (x * z + w * y)
    ny = 2.0 * (y * z - w * x)
    nz = 1.0 - 2.0 * (x * x + y * y)
    return nx, ny, nz


def _tc_body(p_ref, pb_ref, ptb_ref, pt_ref, qt_ref, idx_ref, nrm_ref):
    i = pl.program_id(0)

    p = p_ref[...]
    pt = pt_ref[...]
    sq_r = jnp.sum(p * p, axis=1, keepdims=True)
    sq_c = jnp.sum(pt * pt, axis=0, keepdims=True)
    dots = lax.dot_general(
        pb_ref[...], ptb_ref[...], (((1,), (0,)), ((), ())),
        preferred_element_type=jnp.float32)
    d2 = sq_r + sq_c - 2.0 * dots

    col = lax.broadcasted_iota(jnp.int32, (_R, _N), 1)
    bits = lax.bitcast_convert_type(jnp.maximum(d2, 0.0), jnp.int32)
    keys = lax.bitcast_convert_type(
        (bits & jnp.int32(~0x1FFF)) + (col + jnp.int32(0x00800000)),
        jnp.float32)
    inf = jnp.float32(jnp.inf)

    lane16 = lax.broadcasted_iota(jnp.int32, (_R, 16), 1)
    acc_idx = jnp.zeros((_R, 16), jnp.int32)

    m0 = jnp.min(keys, axis=1, keepdims=True)
    keys = jnp.where(keys == m0, inf, keys)
    for j in range(_K):
        m = jnp.min(keys, axis=1, keepdims=True)
        keys = jnp.where(keys == m, inf, keys)
        mi = lax.bitcast_convert_type(m, jnp.int32) & jnp.int32(0x1FFF)
        acc_idx = jnp.where(lane16 == j, mi, acc_idx)

    idx_ref[...] = acc_idx

    @pl.when(i == 0)
    def _():
        nw, nx_, ny_, nz_ = (qt_ref[0:1, :], qt_ref[1:2, :],
                             qt_ref[2:3, :], qt_ref[3:4, :])
        anx, any_, anz = _normals_from_quat_rows(nw, nx_, ny_, nz_)
        nrm_ref[0:1, :] = anx
        nrm_ref[1:2, :] = any_
        nrm_ref[2:3, :] = anz
        nrm_ref[3:8, :] = jnp.zeros((5, _N), jnp.float32)


def _tc_stage(ppad, pb, ptb, pt, qt):
    grid = _N // _R
    return pl.pallas_call(
        _tc_body,
        grid=(grid,),
        in_specs=[
            pl.BlockSpec((_R, 8), lambda i: (i, 0)),
            pl.BlockSpec((_R, 8), lambda i: (i, 0)),
            pl.BlockSpec((8, _N), lambda i: (0, 0)),
            pl.BlockSpec((8, _N), lambda i: (0, 0)),
            pl.BlockSpec((8, _N), lambda i: (0, 0)),
        ],
        out_specs=[
            pl.BlockSpec((_R, 16), lambda i: (i, 0)),
            pl.BlockSpec((8, _N), lambda i: (0, 0)),
        ],
        out_shape=[
            jax.ShapeDtypeStruct((_N, 16), jnp.int32),
            jax.ShapeDtypeStruct((8, _N), jnp.float32),
        ],
    )(ppad, pb, ptb, pt, qt)


def _sc_stage(idx_all, nrm, opac):
    mesh = plsc.VectorSubcoreMesh(core_axis_name="c", subcore_axis_name="s")

    @functools.partial(
        pl.kernel, mesh=mesh,
        compiler_params=pltpu.CompilerParams(needs_layout_passes=False),
        out_type=jax.ShapeDtypeStruct((_NW, 16), jnp.float32),
        scratch_types=[
            pltpu.VMEM((_RW, 16), jnp.int32),
            pltpu.VMEM((_N,), jnp.float32),
            pltpu.VMEM((_N,), jnp.float32),
            pltpu.VMEM((_N,), jnp.float32),
            pltpu.VMEM((_RW,), jnp.float32),
            pltpu.VMEM((16,), jnp.float32),
        ],
    )
    def sc_loss(idx_hbm, nrm_hbm, op_hbm, out_hbm,
                idx_v, nx_v, ny_v, nz_v, op_v, acc_v):
        c = lax.axis_index("c")
        s = lax.axis_index("s")
        wid = s * 2 + c
        base = pl.multiple_of(wid * _RW, _RW)
        pltpu.sync_copy(idx_hbm.at[pl.ds(base, _RW)], idx_v)
        pltpu.sync_copy(nrm_hbm.at[0], nx_v)
        pltpu.sync_copy(nrm_hbm.at[1], ny_v)
        pltpu.sync_copy(nrm_hbm.at[2], nz_v)
        pltpu.sync_copy(op_hbm.at[pl.ds(base, _RW)], op_v)

        lanes = lax.iota(jnp.int32, 16)
        total = jnp.zeros((16,), jnp.float32)
        for g in range(_G):
            rows = g * 16 + lanes
            own = base + rows
            onx = plsc.load_gather(nx_v, [own])
            ony = plsc.load_gather(ny_v, [own])
            onz = plsc.load_gather(nz_v, [own])
            mx = jnp.zeros((16,), jnp.float32)
            my = jnp.zeros((16,), jnp.float32)
            mz = jnp.zeros((16,), jnp.float32)
            for j in range(_K):
                nid = plsc.load_gather(idx_v, [rows, jnp.full((16,), j, jnp.int32)])
                mx = mx + plsc.load_gather(nx_v, [nid])
                my = my + plsc.load_gather(ny_v, [nid])
                mz = mz + plsc.load_gather(nz_v, [nid])
            cos_sum = onx * mx + ony * my + onz * mz
            opv = op_v[pl.ds(g * 16, 16)]
            total = total + opv * (1.0 - cos_sum * (1.0 / _K))
        acc_v[...] = total
        pltpu.sync_copy(acc_v, out_hbm.at[wid])

    return sc_loss(idx_all, nrm, opac)


def kernel(positions, rotations, opacities, view_matrices):
    del view_matrices
    n = positions.shape[0]
    ppad = jnp.concatenate(
        [positions, jnp.zeros((n, 5), jnp.float32)], axis=1)
    qpad = jnp.concatenate(
        [rotations, jnp.zeros((n, 4), jnp.float32)], axis=1)
    pt = ppad.T
    qt = qpad.T
    pb = ppad.astype(jnp.bfloat16)
    ptb = pt.astype(jnp.bfloat16)

    idx_all, nrm = _tc_stage(ppad, pb, ptb, pt, qt)
    partials = _sc_stage(idx_all, nrm, opacities.reshape(n))
    return jnp.sum(partials) / n


# speedup vs baseline: 147.9129x; 3.0309x over previous
import functools

import jax
import jax.numpy as jnp
from jax import lax
from jax.experimental import pallas as pl
from jax.experimental.pallas import tpu as pltpu
from jax.experimental.pallas import tpu_sc as plsc

_N = 8192
_R = 512
_K = 10
_NW = 32
_RW = _N // _NW
_G = _RW // 16


def _normals_from_quat_rows(w, x, y, z):
    n2 = w * w + x * x + y * y + z * z
    inv = 1.0 / jnp.maximum(jnp.sqrt(n2), 1e-12)
    w, x, y, z = w * inv, x * inv, y * inv, z * inv
    nx = 2.0 * (x * z + w * y)
    ny = 2.0 * (y * z - w * x)
    nz = 1.0 - 2.0 * (x * x + y * y)
    return nx, ny, nz


def _tc_body(p_ref, pb_ref, ptb_ref, pt_ref, qt_ref, idx_ref, nrm_ref):
    i = pl.program_id(0)

    p = p_ref[...]
    pt = pt_ref[...]
    sq_r = jnp.sum(p * p, axis=1, keepdims=True)
    sq_c = jnp.sum(pt * pt, axis=0, keepdims=True)
    dots2 = lax.dot_general(
        pb_ref[...], ptb_ref[...], (((1,), (0,)), ((), ())),
        preferred_element_type=jnp.float32)
    d2 = (sq_r + sq_c) + dots2

    col = lax.broadcasted_iota(jnp.int32, (_R, _N), 1)
    bits = lax.bitcast_convert_type(jnp.maximum(d2, 0.0), jnp.int32)
    keys = lax.bitcast_convert_type(
        (bits & jnp.int32(~0x1FFF)) + (col + jnp.int32(0x00800000)),
        jnp.float32)
    inf = jnp.float32(jnp.inf)

    w = _N
    while w > 512:
        w //= 2
        keys = jnp.minimum(keys[:, :w], keys[:, w:])

    lane16 = lax.broadcasted_iota(jnp.int32, (_R, 16), 1)
    acc_idx = jnp.zeros((_R, 16), jnp.int32)

    m0 = jnp.min(keys, axis=1, keepdims=True)
    keys = jnp.where(keys == m0, inf, keys)
    for j in range(_K):
        m = jnp.min(keys, axis=1, keepdims=True)
        keys = jnp.where(keys == m, inf, keys)
        mi = lax.bitcast_convert_type(m, jnp.int32) & jnp.int32(0x1FFF)
        acc_idx = jnp.where(lane16 == j, mi, acc_idx)

    idx_ref[...] = acc_idx

    @pl.when(i == 0)
    def _():
        nw, nx_, ny_, nz_ = (qt_ref[0:1, :], qt_ref[1:2, :],
                             qt_ref[2:3, :], qt_ref[3:4, :])
        anx, any_, anz = _normals_from_quat_rows(nw, nx_, ny_, nz_)
        nrm_ref[0:1, :] = anx
        nrm_ref[1:2, :] = any_
        nrm_ref[2:3, :] = anz
        nrm_ref[3:8, :] = jnp.zeros((5, _N), jnp.float32)


def _tc_stage(ppad, pb, ptb, pt, qt):
    grid = _N // _R
    return pl.pallas_call(
        _tc_body,
        grid=(grid,),
        in_specs=[
            pl.BlockSpec((_R, 8), lambda i: (i, 0)),
            pl.BlockSpec((_R, 8), lambda i: (i, 0)),
            pl.BlockSpec((8, _N), lambda i: (0, 0)),
            pl.BlockSpec((8, _N), lambda i: (0, 0)),
            pl.BlockSpec((8, _N), lambda i: (0, 0)),
        ],
        out_specs=[
            pl.BlockSpec((_R, 16), lambda i: (i, 0)),
            pl.BlockSpec((8, _N), lambda i: (0, 0)),
        ],
        out_shape=[
            jax.ShapeDtypeStruct((_N, 16), jnp.int32),
            jax.ShapeDtypeStruct((8, _N), jnp.float32),
        ],
    )(ppad, pb, ptb, pt, qt)


def _sc_stage(idx_all, nrm, opac):
    mesh = plsc.VectorSubcoreMesh(core_axis_name="c", subcore_axis_name="s")

    @functools.partial(
        pl.kernel, mesh=mesh,
        compiler_params=pltpu.CompilerParams(needs_layout_passes=False),
        out_type=jax.ShapeDtypeStruct((_NW, 16), jnp.float32),
        scratch_types=[
            pltpu.VMEM((_RW, 16), jnp.int32),
            pltpu.VMEM((_N,), jnp.float32),
            pltpu.VMEM((_N,), jnp.float32),
            pltpu.VMEM((_N,), jnp.float32),
            pltpu.VMEM((_RW,), jnp.float32),
            pltpu.VMEM((16,), jnp.float32),
        ],
    )
    def sc_loss(idx_hbm, nrm_hbm, op_hbm, out_hbm,
                idx_v, nx_v, ny_v, nz_v, op_v, acc_v):
        c = lax.axis_index("c")
        s = lax.axis_index("s")
        wid = s * 2 + c
        base = pl.multiple_of(wid * _RW, _RW)
        pltpu.sync_copy(idx_hbm.at[pl.ds(base, _RW)], idx_v)
        pltpu.sync_copy(nrm_hbm.at[0], nx_v)
        pltpu.sync_copy(nrm_hbm.at[1], ny_v)
        pltpu.sync_copy(nrm_hbm.at[2], nz_v)
        pltpu.sync_copy(op_hbm.at[pl.ds(base, _RW)], op_v)

        lanes = lax.iota(jnp.int32, 16)
        total = jnp.zeros((16,), jnp.float32)
        for g in range(_G):
            rows = g * 16 + lanes
            own = base + rows
            onx = plsc.load_gather(nx_v, [own])
            ony = plsc.load_gather(ny_v, [own])
            onz = plsc.load_gather(nz_v, [own])
            mx = jnp.zeros((16,), jnp.float32)
            my = jnp.zeros((16,), jnp.float32)
            mz = jnp.zeros((16,), jnp.float32)
            for j in range(_K):
                nid = plsc.load_gather(idx_v, [rows, jnp.full((16,), j, jnp.int32)])
                mx = mx + plsc.load_gather(nx_v, [nid])
                my = my + plsc.load_gather(ny_v, [nid])
                mz = mz + plsc.load_gather(nz_v, [nid])
            cos_sum = onx * mx + ony * my + onz * mz
            opv = op_v[pl.ds(g * 16, 16)]
            total = total + opv * (1.0 - cos_sum * (1.0 / _K))
        acc_v[...] = total
        pltpu.sync_copy(acc_v, out_hbm.at[wid])

    return sc_loss(idx_all, nrm, opac)


def kernel(positions, rotations, opacities, view_matrices):
    del view_matrices
    n = positions.shape[0]
    ppad = jnp.concatenate(
        [positions, jnp.zeros((n, 5), jnp.float32)], axis=1)
    qpad = jnp.concatenate(
        [rotations, jnp.zeros((n, 4), jnp.float32)], axis=1)
    pt = ppad.T
    qt = qpad.T
    pb = (-2.0 * ppad).astype(jnp.bfloat16)
    ptb = pt.astype(jnp.bfloat16)

    idx_all, nrm = _tc_stage(ppad, pb, ptb, pt, qt)
    partials = _sc_stage(idx_all, nrm, opacities.reshape(n))
    return jnp.sum(partials) / n


# speedup vs baseline: 156.3065x; 1.0567x over previous
import functools

import jax
import jax.numpy as jnp
from jax import lax
from jax.experimental import pallas as pl
from jax.experimental.pallas import tpu as pltpu
from jax.experimental.pallas import tpu_sc as plsc

_N = 8192
_R = 1024
_K = 10
_NW = 32
_RW = _N // _NW
_G = _RW // 16


def _normals_from_quat_rows(w, x, y, z):
    n2 = w * w + x * x + y * y + z * z
    inv = 1.0 / jnp.maximum(jnp.sqrt(n2), 1e-12)
    w, x, y, z = w * inv, x * inv, y * inv, z * inv
    nx = 2.0 * (x * z + w * y)
    ny = 2.0 * (y * z - w * x)
    nz = 1.0 - 2.0 * (x * x + y * y)
    return nx, ny, nz


def _tc_body(p_ref, pb_ref, ptb_ref, pt_ref, qt_ref, idx_ref, nrm_ref):
    i = pl.program_id(0)

    p = p_ref[...]
    pt = pt_ref[...]
    sq_r = jnp.sum(p * p, axis=1, keepdims=True)
    sq_c = jnp.sum(pt * pt, axis=0, keepdims=True)
    dots2 = lax.dot_general(
        pb_ref[...], ptb_ref[...], (((1,), (0,)), ((), ())),
        preferred_element_type=jnp.float32)
    d2 = (sq_r + sq_c) + dots2

    col = lax.broadcasted_iota(jnp.int32, (_R, _N), 1)
    bits = lax.bitcast_convert_type(jnp.maximum(d2, 0.0), jnp.int32)
    keys = lax.bitcast_convert_type(
        (bits & jnp.int32(~0x1FFF)) + (col + jnp.int32(0x00800000)),
        jnp.float32)
    inf = jnp.float32(jnp.inf)

    w = _N
    while w > 512:
        w //= 2
        keys = jnp.minimum(keys[:, :w], keys[:, w:])

    lane16 = lax.broadcasted_iota(jnp.int32, (_R, 16), 1)
    acc_idx = jnp.zeros((_R, 16), jnp.int32)

    m0 = jnp.min(keys, axis=1, keepdims=True)
    keys = jnp.where(keys == m0, inf, keys)
    for j in range(_K):
        m = jnp.min(keys, axis=1, keepdims=True)
        keys = jnp.where(keys == m, inf, keys)
        mi = lax.bitcast_convert_type(m, jnp.int32) & jnp.int32(0x1FFF)
        acc_idx = jnp.where(lane16 == j, mi, acc_idx)

    idx_ref[...] = acc_idx

    @pl.when(i == 0)
    def _():
        nw, nx_, ny_, nz_ = (qt_ref[0:1, :], qt_ref[1:2, :],
                             qt_ref[2:3, :], qt_ref[3:4, :])
        anx, any_, anz = _normals_from_quat_rows(nw, nx_, ny_, nz_)
        nrm_ref[0:1, :] = anx
        nrm_ref[1:2, :] = any_
        nrm_ref[2:3, :] = anz
        nrm_ref[3:8, :] = jnp.zeros((5, _N), jnp.float32)


def _tc_stage(ppad, pb, ptb, pt, qt):
    grid = _N // _R
    return pl.pallas_call(
        _tc_body,
        grid=(grid,),
        in_specs=[
            pl.BlockSpec((_R, 8), lambda i: (i, 0)),
            pl.BlockSpec((_R, 8), lambda i: (i, 0)),
            pl.BlockSpec((8, _N), lambda i: (0, 0)),
            pl.BlockSpec((8, _N), lambda i: (0, 0)),
            pl.BlockSpec((8, _N), lambda i: (0, 0)),
        ],
        out_specs=[
            pl.BlockSpec((_R, 16), lambda i: (i, 0)),
            pl.BlockSpec((8, _N), lambda i: (0, 0)),
        ],
        out_shape=[
            jax.ShapeDtypeStruct((_N, 16), jnp.int32),
            jax.ShapeDtypeStruct((8, _N), jnp.float32),
        ],
    )(ppad, pb, ptb, pt, qt)


def _sc_stage(idx_all, nrm, opac):
    mesh = plsc.VectorSubcoreMesh(core_axis_name="c", subcore_axis_name="s")

    @functools.partial(
        pl.kernel, mesh=mesh,
        compiler_params=pltpu.CompilerParams(needs_layout_passes=False),
        out_type=jax.ShapeDtypeStruct((_NW, 16), jnp.float32),
        scratch_types=[
            pltpu.VMEM((_RW, 16), jnp.int32),
            pltpu.VMEM((_N,), jnp.float32),
            pltpu.VMEM((_N,), jnp.float32),
            pltpu.VMEM((_N,), jnp.float32),
            pltpu.VMEM((_RW,), jnp.float32),
            pltpu.VMEM((16,), jnp.float32),
        ],
    )
    def sc_loss(idx_hbm, nrm_hbm, op_hbm, out_hbm,
                idx_v, nx_v, ny_v, nz_v, op_v, acc_v):
        c = lax.axis_index("c")
        s = lax.axis_index("s")
        wid = s * 2 + c
        base = pl.multiple_of(wid * _RW, _RW)
        pltpu.sync_copy(idx_hbm.at[pl.ds(base, _RW)], idx_v)
        pltpu.sync_copy(nrm_hbm.at[0], nx_v)
        pltpu.sync_copy(nrm_hbm.at[1], ny_v)
        pltpu.sync_copy(nrm_hbm.at[2], nz_v)
        pltpu.sync_copy(op_hbm.at[pl.ds(base, _RW)], op_v)

        lanes = lax.iota(jnp.int32, 16)
        total = jnp.zeros((16,), jnp.float32)
        for g in range(_G):
            rows = g * 16 + lanes
            own = base + rows
            onx = plsc.load_gather(nx_v, [own])
            ony = plsc.load_gather(ny_v, [own])
            onz = plsc.load_gather(nz_v, [own])
            mx = jnp.zeros((16,), jnp.float32)
            my = jnp.zeros((16,), jnp.float32)
            mz = jnp.zeros((16,), jnp.float32)
            for j in range(_K):
                nid = plsc.load_gather(idx_v, [rows, jnp.full((16,), j, jnp.int32)])
                mx = mx + plsc.load_gather(nx_v, [nid])
                my = my + plsc.load_gather(ny_v, [nid])
                mz = mz + plsc.load_gather(nz_v, [nid])
            cos_sum = onx * mx + ony * my + onz * mz
            opv = op_v[pl.ds(g * 16, 16)]
            total = total + opv * (1.0 - cos_sum * (1.0 / _K))
        acc_v[...] = total
        pltpu.sync_copy(acc_v, out_hbm.at[wid])

    return sc_loss(idx_all, nrm, opac)


def kernel(positions, rotations, opacities, view_matrices):
    del view_matrices
    n = positions.shape[0]
    ppad = jnp.concatenate(
        [positions, jnp.zeros((n, 5), jnp.float32)], axis=1)
    qpad = jnp.concatenate(
        [rotations, jnp.zeros((n, 4), jnp.float32)], axis=1)
    pt = ppad.T
    qt = qpad.T
    pb = (-2.0 * ppad).astype(jnp.bfloat16)
    ptb = pt.astype(jnp.bfloat16)

    idx_all, nrm = _tc_stage(ppad, pb, ptb, pt, qt)
    partials = _sc_stage(idx_all, nrm, opacities.reshape(n))
    return jnp.sum(partials) / n
